# baseline (speedup 1.0000x reference)
import functools

import jax
import jax.numpy as jnp
from jax import lax
from jax.experimental import pallas as pl
from jax.experimental.pallas import tpu as pltpu
from jax.experimental.pallas import tpu_sc as plsc

N_TOK = 2048
TOP_K = 2
N_EXP = 64
D_MODEL = 1024
D_INNER = 512
NK = N_TOK * TOP_K

NUM_WORKERS = 32
TOK_PER_W = N_TOK // NUM_WORKERS
CHUNK = 32
TILE = 128


def _route_kernel(flat_ref, dest_ref, offs_ref):
    flat = flat_ref[:, :]
    e_iota = lax.broadcasted_iota(jnp.int32, (NK, N_EXP), 1)
    oh = (flat == e_iota).astype(jnp.int32)
    cnt = oh
    k = 1
    while k < NK:
        cnt = cnt + jnp.concatenate(
            [jnp.zeros((k, N_EXP), jnp.int32), cnt[: NK - k, :]], axis=0)
        k *= 2
    rank = jnp.sum(cnt * oh, axis=1, keepdims=True) - 1
    totals = cnt[NK - 1 : NK, :]
    incl = totals
    k = 1
    while k < N_EXP:
        incl = incl + jnp.concatenate(
            [jnp.zeros((1, k), jnp.int32), incl[:, : N_EXP - k]], axis=1)
        k *= 2
    excl = incl - totals
    seg_base = jnp.sum(excl * oh, axis=1, keepdims=True)
    dest_ref[:, :] = seg_base + rank
    offs_ref[:, :] = jnp.concatenate(
        [excl, jnp.full((1, 128 - N_EXP), NK, jnp.int32)], axis=1)


def _route(flat2d):
    return pl.pallas_call(
        _route_kernel,
        out_shape=(
            jax.ShapeDtypeStruct((NK, 1), jnp.int32),
            jax.ShapeDtypeStruct((1, 128), jnp.int32),
        ),
    )(flat2d)


@functools.lru_cache(maxsize=None)
def _sc_mesh():
    return plsc.VectorSubcoreMesh(core_axis_name="c", subcore_axis_name="s")


def _dispatch_kernel(x_hbm, d0_hbm, d1_hbm, xs_hbm, idx0_v, idx1_v, rows_v, sem):
    wid = lax.axis_index("s") * 2 + lax.axis_index("c")
    base = wid * TOK_PER_W
    pltpu.sync_copy(d0_hbm.at[pl.ds(base, TOK_PER_W)], idx0_v)
    pltpu.sync_copy(d1_hbm.at[pl.ds(base, TOK_PER_W)], idx1_v)
    pltpu.sync_copy(x_hbm.at[pl.ds(base, TOK_PER_W)], rows_v)
    pltpu.async_copy(rows_v, xs_hbm.at[idx0_v], sem).wait()
    pltpu.async_copy(rows_v, xs_hbm.at[idx1_v], sem).wait()


@functools.lru_cache(maxsize=None)
def _dispatch():
    return pl.kernel(
        _dispatch_kernel,
        out_type=jax.ShapeDtypeStruct((NK, D_MODEL), jnp.float32),
        mesh=_sc_mesh(),
        scratch_types=[
            pltpu.VMEM((TOK_PER_W,), jnp.int32),
            pltpu.VMEM((TOK_PER_W,), jnp.int32),
            pltpu.VMEM((TOK_PER_W, D_MODEL), jnp.float32),
            pltpu.SemaphoreType.DMA,
        ],
    )


def _gmm_kernel(offs_ref, xs_ref, gu_hbm, dn_hbm, out_ref,
                gu_buf, dn_buf, gu_bf_buf, dn_bf_buf, sem_gu, sem_dn):
    def copies(e, slot):
        cg = pltpu.make_async_copy(gu_hbm.at[e], gu_buf.at[slot], sem_gu)
        cd = pltpu.make_async_copy(dn_hbm.at[e], dn_buf.at[slot], sem_dn)
        return cg, cd

    cg0, cd0 = copies(0, 0)
    cg0.start()
    cd0.start()

    def expert_step(e, _):
        slot = lax.rem(e, 2)

        @pl.when(e + 1 < N_EXP)
        def _():
            cgn, cdn = copies(e + 1, 1 - slot)
            cgn.start()
            cdn.start()

        cgw, cdw = copies(e, slot)
        cgw.wait()
        cdw.wait()
        gu_bf_buf[...] = gu_buf[slot].astype(jnp.bfloat16)
        dn_bf_buf[...] = dn_buf[slot].astype(jnp.bfloat16)

        start = offs_ref[e]
        end = offs_ref[e + 1]
        t0 = start // TILE
        t1 = lax.div(end + TILE - 1, TILE)

        def body(t, _):
            r = t * TILE
            x_tile = xs_ref[pl.ds(r, TILE), :].astype(jnp.bfloat16)
            h = jnp.dot(x_tile, gu_bf_buf[...],
                        preferred_element_type=jnp.float32)
            g = h[:, :D_INNER]
            u = h[:, D_INNER:]
            act = g * (1.0 / (1.0 + jnp.exp(-g))) * u
            o = jnp.dot(act.astype(jnp.bfloat16), dn_bf_buf[...],
                        preferred_element_type=jnp.float32)
            rows = r + lax.broadcasted_iota(jnp.int32, (TILE, 1), 0)
            mask = (rows >= start) & (rows < end)
            old = out_ref[pl.ds(r, TILE), :]
            out_ref[pl.ds(r, TILE), :] = jnp.where(mask, o, old)
            return 0

        lax.fori_loop(t0, t1, body, 0)
        return 0

    lax.fori_loop(0, N_EXP, expert_step, 0)


def _gmm(offs1d, x_sorted, gate_up_proj, down_proj):
    return pl.pallas_call(
        _gmm_kernel,
        in_specs=[
            pl.BlockSpec(memory_space=pltpu.SMEM),
            pl.BlockSpec(memory_space=pltpu.VMEM),
            pl.BlockSpec(memory_space=pl.ANY),
            pl.BlockSpec(memory_space=pl.ANY),
        ],
        out_specs=pl.BlockSpec(memory_space=pltpu.VMEM),
        out_shape=jax.ShapeDtypeStruct((NK, D_MODEL), jnp.float32),
        scratch_shapes=[
            pltpu.VMEM((2, D_MODEL, 2 * D_INNER), jnp.float32),
            pltpu.VMEM((2, D_INNER, D_MODEL), jnp.float32),
            pltpu.VMEM((D_MODEL, 2 * D_INNER), jnp.bfloat16),
            pltpu.VMEM((D_INNER, D_MODEL), jnp.bfloat16),
            pltpu.SemaphoreType.DMA,
            pltpu.SemaphoreType.DMA,
        ],
    )(offs1d, x_sorted, gate_up_proj, down_proj)


def _combine_kernel(os_hbm, d0_hbm, d1_hbm, w0_hbm, w1_hbm, out_hbm,
                    idx0_v, idx1_v, w0_v, w1_v, r0_v, r1_v, o_v, sem):
    wid = lax.axis_index("s") * 2 + lax.axis_index("c")
    tbase = wid * TOK_PER_W
    pltpu.sync_copy(d0_hbm.at[pl.ds(tbase, TOK_PER_W)], idx0_v)
    pltpu.sync_copy(d1_hbm.at[pl.ds(tbase, TOK_PER_W)], idx1_v)
    pltpu.sync_copy(w0_hbm.at[pl.ds(tbase, TOK_PER_W)], w0_v)
    pltpu.sync_copy(w1_hbm.at[pl.ds(tbase, TOK_PER_W)], w1_v)

    def chunk(ci, _):
        coff = ci * CHUNK
        cp0 = pltpu.async_copy(os_hbm.at[idx0_v.at[pl.ds(coff, CHUNK)]], r0_v, sem)
        cp1 = pltpu.async_copy(os_hbm.at[idx1_v.at[pl.ds(coff, CHUNK)]], r1_v, sem)
        cp0.wait()
        cp1.wait()

        def tok_group(g, _):
            wv0 = w0_v[pl.ds(coff + g * 16, 16)]
            wv1 = w1_v[pl.ds(coff + g * 16, 16)]
            for lane in range(16):
                a = wv0[lane]
                b = wv1[lane]
                t = g * 16 + lane

                @plsc.parallel_loop(0, D_MODEL // 16, unroll=8)
                def col(j, a=a, b=b, t=t):
                    sl = pl.ds(j * 16, 16)
                    o_v[t, sl] = a * r0_v[t, sl] + b * r1_v[t, sl]
            return 0

        lax.fori_loop(0, CHUNK // 16, tok_group, 0)
        pltpu.sync_copy(o_v, out_hbm.at[pl.ds(tbase + coff, CHUNK)])
        return 0

    lax.fori_loop(0, TOK_PER_W // CHUNK, chunk, 0)


@functools.lru_cache(maxsize=None)
def _combine():
    return pl.kernel(
        _combine_kernel,
        out_type=jax.ShapeDtypeStruct((N_TOK, D_MODEL), jnp.float32),
        mesh=_sc_mesh(),
        scratch_types=[
            pltpu.VMEM((TOK_PER_W,), jnp.int32),
            pltpu.VMEM((TOK_PER_W,), jnp.int32),
            pltpu.VMEM((TOK_PER_W,), jnp.float32),
            pltpu.VMEM((TOK_PER_W,), jnp.float32),
            pltpu.VMEM((CHUNK, D_MODEL), jnp.float32),
            pltpu.VMEM((CHUNK, D_MODEL), jnp.float32),
            pltpu.VMEM((CHUNK, D_MODEL), jnp.float32),
            pltpu.SemaphoreType.DMA,
        ],
    )


def kernel(x, topk_indices, topk_weights, gate_up_proj, down_proj):
    flat2d = topk_indices.astype(jnp.int32).reshape(NK, 1)
    dest, offs = _route(flat2d)
    dest_nk = dest.reshape(N_TOK, TOP_K)
    d0 = dest_nk[:, 0]
    d1 = dest_nk[:, 1]
    w0 = topk_weights[:, 0]
    w1 = topk_weights[:, 1]
    offs1d = offs.reshape(128)

    x_sorted = _dispatch()(x, d0, d1)
    out_sorted = _gmm(offs1d, x_sorted, gate_up_proj, down_proj)
    return _combine()(out_sorted, d0, d1, w0, w1)


# speedup vs baseline: 1.0089x; 1.0089x over previous
import functools

import jax
import jax.numpy as jnp
from jax import lax
from jax.experimental import pallas as pl
from jax.experimental.pallas import tpu as pltpu
from jax.experimental.pallas import tpu_sc as plsc

N_TOK = 2048
TOP_K = 2
N_EXP = 64
D_MODEL = 1024
D_INNER = 512
NK = N_TOK * TOP_K

NUM_WORKERS = 32
TOK_PER_W = N_TOK // NUM_WORKERS
CHUNK = 32
TILE = 256


def _route_kernel(flat_ref, dest_ref, offs_ref):
    flat = flat_ref[:, :]
    e_iota = lax.broadcasted_iota(jnp.int32, (NK, N_EXP), 1)
    oh = (flat == e_iota).astype(jnp.int32)
    cnt = oh
    k = 1
    while k < NK:
        cnt = cnt + jnp.concatenate(
            [jnp.zeros((k, N_EXP), jnp.int32), cnt[: NK - k, :]], axis=0)
        k *= 2
    rank = jnp.sum(cnt * oh, axis=1, keepdims=True) - 1
    totals = cnt[NK - 1 : NK, :]
    incl = totals
    k = 1
    while k < N_EXP:
        incl = incl + jnp.concatenate(
            [jnp.zeros((1, k), jnp.int32), incl[:, : N_EXP - k]], axis=1)
        k *= 2
    excl = incl - totals
    seg_base = jnp.sum(excl * oh, axis=1, keepdims=True)
    dest_ref[:, :] = seg_base + rank
    offs_ref[:, :] = jnp.concatenate(
        [excl, jnp.full((1, 128 - N_EXP), NK, jnp.int32)], axis=1)


def _route(flat2d):
    return pl.pallas_call(
        _route_kernel,
        out_shape=(
            jax.ShapeDtypeStruct((NK, 1), jnp.int32),
            jax.ShapeDtypeStruct((1, 128), jnp.int32),
        ),
    )(flat2d)


@functools.lru_cache(maxsize=None)
def _sc_mesh():
    return plsc.VectorSubcoreMesh(core_axis_name="c", subcore_axis_name="s")


def _dispatch_kernel(x_hbm, d0_hbm, d1_hbm, xs_hbm, idx0_v, idx1_v, rows_v, sem):
    wid = lax.axis_index("s") * 2 + lax.axis_index("c")
    base = wid * TOK_PER_W
    pltpu.sync_copy(d0_hbm.at[pl.ds(base, TOK_PER_W)], idx0_v)
    pltpu.sync_copy(d1_hbm.at[pl.ds(base, TOK_PER_W)], idx1_v)
    pltpu.sync_copy(x_hbm.at[pl.ds(base, TOK_PER_W)], rows_v)
    pltpu.async_copy(rows_v, xs_hbm.at[idx0_v], sem).wait()
    pltpu.async_copy(rows_v, xs_hbm.at[idx1_v], sem).wait()


@functools.lru_cache(maxsize=None)
def _dispatch():
    return pl.kernel(
        _dispatch_kernel,
        out_type=jax.ShapeDtypeStruct((NK, D_MODEL), jnp.float32),
        mesh=_sc_mesh(),
        scratch_types=[
            pltpu.VMEM((TOK_PER_W,), jnp.int32),
            pltpu.VMEM((TOK_PER_W,), jnp.int32),
            pltpu.VMEM((TOK_PER_W, D_MODEL), jnp.float32),
            pltpu.SemaphoreType.DMA,
        ],
    )


def _gmm_kernel(offs_ref, xs_ref, gu_hbm, dn_hbm, out_ref,
                gu_buf, dn_buf, sem_gu, sem_dn):
    def copies(e, slot):
        cg = pltpu.make_async_copy(gu_hbm.at[e], gu_buf.at[slot], sem_gu)
        cd = pltpu.make_async_copy(dn_hbm.at[e], dn_buf.at[slot], sem_dn)
        return cg, cd

    cg0, cd0 = copies(0, 0)
    cg0.start()
    cd0.start()

    def expert_step(e, _):
        slot = lax.rem(e, 2)

        @pl.when(e + 1 < N_EXP)
        def _():
            cgn, cdn = copies(e + 1, 1 - slot)
            cgn.start()
            cdn.start()

        cgw, cdw = copies(e, slot)
        cgw.wait()
        cdw.wait()

        start = offs_ref[e]
        end = offs_ref[e + 1]
        t0 = start // TILE
        t1 = lax.div(end + TILE - 1, TILE)

        def body(t, _):
            r = t * TILE
            x_tile = xs_ref[pl.ds(r, TILE), :].astype(jnp.bfloat16)
            gu_bf = gu_buf[slot].astype(jnp.bfloat16)
            h = jnp.dot(x_tile, gu_bf, preferred_element_type=jnp.float32)
            g = h[:, :D_INNER]
            u = h[:, D_INNER:]
            act = g * (1.0 / (1.0 + jnp.exp(-g))) * u
            dn_bf = dn_buf[slot].astype(jnp.bfloat16)
            o = jnp.dot(act.astype(jnp.bfloat16), dn_bf,
                        preferred_element_type=jnp.float32)
            rows = r + lax.broadcasted_iota(jnp.int32, (TILE, 1), 0)
            mask = (rows >= start) & (rows < end)
            old = out_ref[pl.ds(r, TILE), :]
            out_ref[pl.ds(r, TILE), :] = jnp.where(mask, o, old)
            return 0

        lax.fori_loop(t0, t1, body, 0)
        return 0

    lax.fori_loop(0, N_EXP, expert_step, 0)


def _gmm(offs1d, x_sorted, gate_up_proj, down_proj):
    return pl.pallas_call(
        _gmm_kernel,
        in_specs=[
            pl.BlockSpec(memory_space=pltpu.SMEM),
            pl.BlockSpec(memory_space=pltpu.VMEM),
            pl.BlockSpec(memory_space=pl.ANY),
            pl.BlockSpec(memory_space=pl.ANY),
        ],
        out_specs=pl.BlockSpec(memory_space=pltpu.VMEM),
        out_shape=jax.ShapeDtypeStruct((NK, D_MODEL), jnp.float32),
        scratch_shapes=[
            pltpu.VMEM((2, D_MODEL, 2 * D_INNER), jnp.float32),
            pltpu.VMEM((2, D_INNER, D_MODEL), jnp.float32),
            pltpu.SemaphoreType.DMA,
            pltpu.SemaphoreType.DMA,
        ],
    )(offs1d, x_sorted, gate_up_proj, down_proj)


def _combine_kernel(os_hbm, d0_hbm, d1_hbm, w0_hbm, w1_hbm, out_hbm,
                    idx0_v, idx1_v, w0_v, w1_v, r0_v, r1_v, o_v, sem):
    wid = lax.axis_index("s") * 2 + lax.axis_index("c")
    tbase = wid * TOK_PER_W
    pltpu.sync_copy(d0_hbm.at[pl.ds(tbase, TOK_PER_W)], idx0_v)
    pltpu.sync_copy(d1_hbm.at[pl.ds(tbase, TOK_PER_W)], idx1_v)
    pltpu.sync_copy(w0_hbm.at[pl.ds(tbase, TOK_PER_W)], w0_v)
    pltpu.sync_copy(w1_hbm.at[pl.ds(tbase, TOK_PER_W)], w1_v)

    def chunk(ci, _):
        coff = ci * CHUNK
        cp0 = pltpu.async_copy(os_hbm.at[idx0_v.at[pl.ds(coff, CHUNK)]], r0_v, sem)
        cp1 = pltpu.async_copy(os_hbm.at[idx1_v.at[pl.ds(coff, CHUNK)]], r1_v, sem)
        cp0.wait()
        cp1.wait()

        def tok_group(g, _):
            wv0 = w0_v[pl.ds(coff + g * 16, 16)]
            wv1 = w1_v[pl.ds(coff + g * 16, 16)]
            for lane in range(16):
                a = wv0[lane]
                b = wv1[lane]
                t = g * 16 + lane

                @plsc.parallel_loop(0, D_MODEL // 16, unroll=8)
                def col(j, a=a, b=b, t=t):
                    sl = pl.ds(j * 16, 16)
                    o_v[t, sl] = a * r0_v[t, sl] + b * r1_v[t, sl]
            return 0

        lax.fori_loop(0, CHUNK // 16, tok_group, 0)
        pltpu.sync_copy(o_v, out_hbm.at[pl.ds(tbase + coff, CHUNK)])
        return 0

    lax.fori_loop(0, TOK_PER_W // CHUNK, chunk, 0)


@functools.lru_cache(maxsize=None)
def _combine():
    return pl.kernel(
        _combine_kernel,
        out_type=jax.ShapeDtypeStruct((N_TOK, D_MODEL), jnp.float32),
        mesh=_sc_mesh(),
        scratch_types=[
            pltpu.VMEM((TOK_PER_W,), jnp.int32),
            pltpu.VMEM((TOK_PER_W,), jnp.int32),
            pltpu.VMEM((TOK_PER_W,), jnp.float32),
            pltpu.VMEM((TOK_PER_W,), jnp.float32),
            pltpu.VMEM((CHUNK, D_MODEL), jnp.float32),
            pltpu.VMEM((CHUNK, D_MODEL), jnp.float32),
            pltpu.VMEM((CHUNK, D_MODEL), jnp.float32),
            pltpu.SemaphoreType.DMA,
        ],
    )


def kernel(x, topk_indices, topk_weights, gate_up_proj, down_proj):
    flat2d = topk_indices.astype(jnp.int32).reshape(NK, 1)
    dest, offs = _route(flat2d)
    dest_nk = dest.reshape(N_TOK, TOP_K)
    d0 = dest_nk[:, 0]
    d1 = dest_nk[:, 1]
    w0 = topk_weights[:, 0]
    w1 = topk_weights[:, 1]
    offs1d = offs.reshape(128)

    x_sorted = _dispatch()(x, d0, d1)
    out_sorted = _gmm(offs1d, x_sorted, gate_up_proj, down_proj)
    return _combine()(out_sorted, d0, d1, w0, w1)


# speedup vs baseline: 1.0610x; 1.0516x over previous
import functools

import jax
import jax.numpy as jnp
from jax import lax
from jax.experimental import pallas as pl
from jax.experimental.pallas import tpu as pltpu
from jax.experimental.pallas import tpu_sc as plsc

N_TOK = 2048
TOP_K = 2
N_EXP = 64
D_MODEL = 1024
D_INNER = 512
NK = N_TOK * TOP_K

NUM_WORKERS = 32
TOK_PER_W = N_TOK // NUM_WORKERS
CHUNK = 16
TILE = 128


def _route_kernel(flat_ref, dest_ref, offs_ref):
    flat = flat_ref[:, :]
    e_iota = lax.broadcasted_iota(jnp.int32, (NK, N_EXP), 1)
    oh = (flat == e_iota).astype(jnp.int32)
    cnt = oh
    k = 1
    while k < NK:
        cnt = cnt + jnp.concatenate(
            [jnp.zeros((k, N_EXP), jnp.int32), cnt[: NK - k, :]], axis=0)
        k *= 2
    rank = jnp.sum(cnt * oh, axis=1, keepdims=True) - 1
    totals = cnt[NK - 1 : NK, :]
    incl = totals
    k = 1
    while k < N_EXP:
        incl = incl + jnp.concatenate(
            [jnp.zeros((1, k), jnp.int32), incl[:, : N_EXP - k]], axis=1)
        k *= 2
    excl = incl - totals
    seg_base = jnp.sum(excl * oh, axis=1, keepdims=True)
    dest_ref[:, :] = seg_base + rank
    offs_ref[:, :] = jnp.concatenate(
        [excl, jnp.full((1, 128 - N_EXP), NK, jnp.int32)], axis=1)


def _route(flat2d):
    return pl.pallas_call(
        _route_kernel,
        out_shape=(
            jax.ShapeDtypeStruct((NK, 1), jnp.int32),
            jax.ShapeDtypeStruct((1, 128), jnp.int32),
        ),
    )(flat2d)


@functools.lru_cache(maxsize=None)
def _sc_mesh():
    return plsc.VectorSubcoreMesh(core_axis_name="c", subcore_axis_name="s")


H_TOK = TOK_PER_W // 2


def _dispatch_kernel(x_hbm, d0_hbm, d1_hbm, xs_hbm,
                     i0a, i0b, i1a, i1b, rows_v, sem_x, sem_s):
    wid = lax.axis_index("s") * 2 + lax.axis_index("c")
    base = wid * TOK_PER_W
    pltpu.sync_copy(d0_hbm.at[pl.ds(base, H_TOK)], i0a)
    pltpu.sync_copy(d0_hbm.at[pl.ds(base + H_TOK, H_TOK)], i0b)
    pltpu.sync_copy(d1_hbm.at[pl.ds(base, H_TOK)], i1a)
    pltpu.sync_copy(d1_hbm.at[pl.ds(base + H_TOK, H_TOK)], i1b)
    cpa = pltpu.async_copy(x_hbm.at[pl.ds(base, H_TOK)], rows_v.at[0], sem_x)
    cpb = pltpu.async_copy(x_hbm.at[pl.ds(base + H_TOK, H_TOK)], rows_v.at[1],
                           sem_x)
    cpa.wait()
    s0 = pltpu.async_copy(rows_v.at[0], xs_hbm.at[i0a], sem_s)
    s1 = pltpu.async_copy(rows_v.at[0], xs_hbm.at[i1a], sem_s)
    cpb.wait()
    s2 = pltpu.async_copy(rows_v.at[1], xs_hbm.at[i0b], sem_s)
    s3 = pltpu.async_copy(rows_v.at[1], xs_hbm.at[i1b], sem_s)
    s0.wait()
    s1.wait()
    s2.wait()
    s3.wait()


@functools.lru_cache(maxsize=None)
def _dispatch():
    return pl.kernel(
        _dispatch_kernel,
        out_type=jax.ShapeDtypeStruct((NK, D_MODEL), jnp.float32),
        mesh=_sc_mesh(),
        scratch_types=[
            pltpu.VMEM((H_TOK,), jnp.int32),
            pltpu.VMEM((H_TOK,), jnp.int32),
            pltpu.VMEM((H_TOK,), jnp.int32),
            pltpu.VMEM((H_TOK,), jnp.int32),
            pltpu.VMEM((2, H_TOK, D_MODEL), jnp.float32),
            pltpu.SemaphoreType.DMA,
            pltpu.SemaphoreType.DMA,
        ],
    )


def _gmm_kernel(offs_ref, xs_ref, gu_hbm, dn_hbm, out_ref,
                gu_buf, dn_buf, sem_gu, sem_dn):
    def copies(e, slot):
        cg = pltpu.make_async_copy(gu_hbm.at[e], gu_buf.at[slot], sem_gu)
        cd = pltpu.make_async_copy(dn_hbm.at[e], dn_buf.at[slot], sem_dn)
        return cg, cd

    cg0, cd0 = copies(0, 0)
    cg0.start()
    cd0.start()

    def expert_step(e, _):
        slot = lax.rem(e, 2)

        @pl.when(e + 1 < N_EXP)
        def _():
            cgn, cdn = copies(e + 1, 1 - slot)
            cgn.start()
            cdn.start()

        cgw, cdw = copies(e, slot)
        cgw.wait()
        cdw.wait()

        start = offs_ref[e]
        end = offs_ref[e + 1]
        t0 = start // TILE
        t1 = lax.div(end + TILE - 1, TILE)

        def body(t, _):
            r = t * TILE
            x_tile = xs_ref[pl.ds(r, TILE), :].astype(jnp.bfloat16)
            gu_bf = gu_buf[slot].astype(jnp.bfloat16)
            h = jnp.dot(x_tile, gu_bf, preferred_element_type=jnp.float32)
            g = h[:, :D_INNER]
            u = h[:, D_INNER:]
            act = g * (1.0 / (1.0 + jnp.exp(-g))) * u
            dn_bf = dn_buf[slot].astype(jnp.bfloat16)
            o = jnp.dot(act.astype(jnp.bfloat16), dn_bf,
                        preferred_element_type=jnp.float32)
            rows = r + lax.broadcasted_iota(jnp.int32, (TILE, 1), 0)
            mask = (rows >= start) & (rows < end)
            old = out_ref[pl.ds(r, TILE), :]
            out_ref[pl.ds(r, TILE), :] = jnp.where(mask, o, old)
            return 0

        lax.fori_loop(t0, t1, body, 0)
        return 0

    lax.fori_loop(0, N_EXP, expert_step, 0)


def _gmm(offs1d, x_sorted, gate_up_proj, down_proj):
    return pl.pallas_call(
        _gmm_kernel,
        in_specs=[
            pl.BlockSpec(memory_space=pltpu.SMEM),
            pl.BlockSpec(memory_space=pltpu.VMEM),
            pl.BlockSpec(memory_space=pl.ANY),
            pl.BlockSpec(memory_space=pl.ANY),
        ],
        out_specs=pl.BlockSpec(memory_space=pltpu.VMEM),
        out_shape=jax.ShapeDtypeStruct((NK, D_MODEL), jnp.float32),
        scratch_shapes=[
            pltpu.VMEM((2, D_MODEL, 2 * D_INNER), jnp.float32),
            pltpu.VMEM((2, D_INNER, D_MODEL), jnp.float32),
            pltpu.SemaphoreType.DMA,
            pltpu.SemaphoreType.DMA,
        ],
    )(offs1d, x_sorted, gate_up_proj, down_proj)


N_CHUNK = TOK_PER_W // CHUNK


def _combine_kernel(os_hbm, d0_hbm, d1_hbm, w0_hbm, w1_hbm, out_hbm,
                    idx0_v, idx1_v, w0_v, w1_v, r0_v, r1_v, o_v, sem_g, sem_o):
    wid = lax.axis_index("s") * 2 + lax.axis_index("c")
    tbase = wid * TOK_PER_W
    pltpu.sync_copy(d0_hbm.at[pl.ds(tbase, TOK_PER_W)], idx0_v)
    pltpu.sync_copy(d1_hbm.at[pl.ds(tbase, TOK_PER_W)], idx1_v)
    pltpu.sync_copy(w0_hbm.at[pl.ds(tbase, TOK_PER_W)], w0_v)
    pltpu.sync_copy(w1_hbm.at[pl.ds(tbase, TOK_PER_W)], w1_v)

    def gather_descs(ci, s):
        c0 = pltpu.make_async_copy(
            os_hbm.at[idx0_v.at[pl.ds(ci * CHUNK, CHUNK)]], r0_v.at[s], sem_g)
        c1 = pltpu.make_async_copy(
            os_hbm.at[idx1_v.at[pl.ds(ci * CHUNK, CHUNK)]], r1_v.at[s], sem_g)
        return c0, c1

    def out_desc(ci, s):
        return pltpu.make_async_copy(
            o_v.at[s], out_hbm.at[pl.ds(tbase + ci * CHUNK, CHUNK)], sem_o)

    g0, g1 = gather_descs(0, 0)
    g0.start()
    g1.start()

    def chunk(ci, _):
        s = lax.rem(ci, 2)

        @pl.when(ci + 1 < N_CHUNK)
        def _():
            n0, n1 = gather_descs(ci + 1, 1 - s)
            n0.start()
            n1.start()

        w0d, w1d = gather_descs(ci, s)
        w0d.wait()
        w1d.wait()

        @pl.when(ci >= 2)
        def _():
            out_desc(ci - 2, s).wait()

        def tok_group(g, _):
            wv0 = w0_v[pl.ds(ci * CHUNK + g * 16, 16)]
            wv1 = w1_v[pl.ds(ci * CHUNK + g * 16, 16)]
            for lane in range(16):
                a = wv0[lane]
                b = wv1[lane]
                t = g * 16 + lane

                @plsc.parallel_loop(0, D_MODEL // 16, unroll=8)
                def col(j, a=a, b=b, t=t, s=s):
                    sl = pl.ds(j * 16, 16)
                    o_v[s, t, sl] = a * r0_v[s, t, sl] + b * r1_v[s, t, sl]
            return 0

        lax.fori_loop(0, CHUNK // 16, tok_group, 0)
        out_desc(ci, s).start()
        return 0

    lax.fori_loop(0, N_CHUNK, chunk, 0)
    out_desc(N_CHUNK - 2, lax.rem(N_CHUNK - 2, 2)).wait()
    out_desc(N_CHUNK - 1, lax.rem(N_CHUNK - 1, 2)).wait()


@functools.lru_cache(maxsize=None)
def _combine():
    return pl.kernel(
        _combine_kernel,
        out_type=jax.ShapeDtypeStruct((N_TOK, D_MODEL), jnp.float32),
        mesh=_sc_mesh(),
        scratch_types=[
            pltpu.VMEM((TOK_PER_W,), jnp.int32),
            pltpu.VMEM((TOK_PER_W,), jnp.int32),
            pltpu.VMEM((TOK_PER_W,), jnp.float32),
            pltpu.VMEM((TOK_PER_W,), jnp.float32),
            pltpu.VMEM((2, CHUNK, D_MODEL), jnp.float32),
            pltpu.VMEM((2, CHUNK, D_MODEL), jnp.float32),
            pltpu.VMEM((2, CHUNK, D_MODEL), jnp.float32),
            pltpu.SemaphoreType.DMA,
            pltpu.SemaphoreType.DMA,
        ],
    )


def kernel(x, topk_indices, topk_weights, gate_up_proj, down_proj):
    flat2d = topk_indices.astype(jnp.int32).reshape(NK, 1)
    dest, offs = _route(flat2d)
    dest_nk = dest.reshape(N_TOK, TOP_K)
    d0 = dest_nk[:, 0]
    d1 = dest_nk[:, 1]
    w0 = topk_weights[:, 0]
    w1 = topk_weights[:, 1]
    offs1d = offs.reshape(128)

    x_sorted = _dispatch()(x, d0, d1)
    out_sorted = _gmm(offs1d, x_sorted, gate_up_proj, down_proj)
    return _combine()(out_sorted, d0, d1, w0, w1)


# speedup vs baseline: 1.0684x; 1.0070x over previous
import functools

import jax
import jax.numpy as jnp
from jax import lax
from jax.experimental import pallas as pl
from jax.experimental.pallas import tpu as pltpu
from jax.experimental.pallas import tpu_sc as plsc

N_TOK = 2048
TOP_K = 2
N_EXP = 64
D_MODEL = 1024
D_INNER = 512
NK = N_TOK * TOP_K

NUM_WORKERS = 32
TOK_PER_W = N_TOK // NUM_WORKERS
CHUNK = 16
TILE = 128


def _route_kernel(flat_ref, dest_ref, offs_ref):
    flat = flat_ref[:, :]
    e_iota = lax.broadcasted_iota(jnp.int32, (NK, N_EXP), 1)
    oh = (flat == e_iota).astype(jnp.int32)
    cnt = oh
    k = 1
    while k < NK:
        cnt = cnt + jnp.concatenate(
            [jnp.zeros((k, N_EXP), jnp.int32), cnt[: NK - k, :]], axis=0)
        k *= 2
    rank = jnp.sum(cnt * oh, axis=1, keepdims=True) - 1
    totals = cnt[NK - 1 : NK, :]
    incl = totals
    k = 1
    while k < N_EXP:
        incl = incl + jnp.concatenate(
            [jnp.zeros((1, k), jnp.int32), incl[:, : N_EXP - k]], axis=1)
        k *= 2
    excl = incl - totals
    seg_base = jnp.sum(excl * oh, axis=1, keepdims=True)
    dest_ref[:, :] = seg_base + rank
    offs_ref[:, :] = jnp.concatenate(
        [excl, jnp.full((1, 128 - N_EXP), NK, jnp.int32)], axis=1)


def _route(flat2d):
    return pl.pallas_call(
        _route_kernel,
        out_shape=(
            jax.ShapeDtypeStruct((NK, 1), jnp.int32),
            jax.ShapeDtypeStruct((1, 128), jnp.int32),
        ),
    )(flat2d)


@functools.lru_cache(maxsize=None)
def _sc_mesh():
    return plsc.VectorSubcoreMesh(core_axis_name="c", subcore_axis_name="s")


H_TOK = TOK_PER_W // 2


def _dispatch_kernel(x_hbm, d0_hbm, d1_hbm, xs_hbm,
                     i0a, i0b, i1a, i1b, rows_v, sem_x, sem_s):
    wid = lax.axis_index("s") * 2 + lax.axis_index("c")
    base = wid * TOK_PER_W
    pltpu.sync_copy(d0_hbm.at[pl.ds(base, H_TOK)], i0a)
    pltpu.sync_copy(d0_hbm.at[pl.ds(base + H_TOK, H_TOK)], i0b)
    pltpu.sync_copy(d1_hbm.at[pl.ds(base, H_TOK)], i1a)
    pltpu.sync_copy(d1_hbm.at[pl.ds(base + H_TOK, H_TOK)], i1b)
    cpa = pltpu.async_copy(x_hbm.at[pl.ds(base, H_TOK)], rows_v.at[0], sem_x)
    cpb = pltpu.async_copy(x_hbm.at[pl.ds(base + H_TOK, H_TOK)], rows_v.at[1],
                           sem_x)
    cpa.wait()
    s0 = pltpu.async_copy(rows_v.at[0], xs_hbm.at[i0a], sem_s)
    s1 = pltpu.async_copy(rows_v.at[0], xs_hbm.at[i1a], sem_s)
    cpb.wait()
    s2 = pltpu.async_copy(rows_v.at[1], xs_hbm.at[i0b], sem_s)
    s3 = pltpu.async_copy(rows_v.at[1], xs_hbm.at[i1b], sem_s)
    s0.wait()
    s1.wait()
    s2.wait()
    s3.wait()


@functools.lru_cache(maxsize=None)
def _dispatch():
    return pl.kernel(
        _dispatch_kernel,
        out_type=jax.ShapeDtypeStruct((NK, D_MODEL), jnp.float32),
        mesh=_sc_mesh(),
        scratch_types=[
            pltpu.VMEM((H_TOK,), jnp.int32),
            pltpu.VMEM((H_TOK,), jnp.int32),
            pltpu.VMEM((H_TOK,), jnp.int32),
            pltpu.VMEM((H_TOK,), jnp.int32),
            pltpu.VMEM((2, H_TOK, D_MODEL), jnp.float32),
            pltpu.SemaphoreType.DMA,
            pltpu.SemaphoreType.DMA,
        ],
    )


def _gmm_kernel(offs_ref, xs_hbm, gu_hbm, dn_hbm, out_hbm,
                xs_buf, out_buf, gu_buf, dn_buf,
                sem_xs, sem_out, sem_gu, sem_dn):
    def copies(e, slot):
        cg = pltpu.make_async_copy(gu_hbm.at[e], gu_buf.at[slot], sem_gu)
        cd = pltpu.make_async_copy(dn_hbm.at[e], dn_buf.at[slot], sem_dn)
        return cg, cd

    def out_tile_desc(t):
        return pltpu.make_async_copy(
            out_buf.at[pl.ds(t * TILE, TILE)],
            out_hbm.at[pl.ds(t * TILE, TILE)], sem_out)

    xs_cp = pltpu.make_async_copy(xs_hbm, xs_buf, sem_xs)
    xs_cp.start()
    cg0, cd0 = copies(0, 0)
    cg0.start()
    cd0.start()
    cg1, cd1 = copies(1, 1)
    cg1.start()
    cd1.start()
    xs_cp.wait()

    def expert_step(e, done):
        slot = lax.rem(e, 2)
        cgw, cdw = copies(e, slot)
        cgw.wait()
        cdw.wait()

        start = offs_ref[e]
        end = offs_ref[e + 1]
        t0 = start // TILE
        t1 = lax.div(end + TILE - 1, TILE)

        def body(t, _):
            r = t * TILE
            x_tile = xs_buf[pl.ds(r, TILE), :].astype(jnp.bfloat16)
            gu_bf = gu_buf[slot].astype(jnp.bfloat16)
            h = jnp.dot(x_tile, gu_bf, preferred_element_type=jnp.float32)
            g = h[:, :D_INNER]
            u = h[:, D_INNER:]
            act = g * (1.0 / (1.0 + jnp.exp(-g))) * u
            dn_bf = dn_buf[slot].astype(jnp.bfloat16)
            o = jnp.dot(act.astype(jnp.bfloat16), dn_bf,
                        preferred_element_type=jnp.float32)
            rows = r + lax.broadcasted_iota(jnp.int32, (TILE, 1), 0)
            mask = (rows >= start) & (rows < end)
            old = out_buf[pl.ds(r, TILE), :]
            out_buf[pl.ds(r, TILE), :] = jnp.where(mask, o, old)
            return 0

        lax.fori_loop(t0, t1, body, 0)

        @pl.when(e + 2 < N_EXP)
        def _():
            cgn, cdn = copies(e + 2, slot)
            cgn.start()
            cdn.start()

        next_done = offs_ref[e + 1] // TILE

        def ship(t, _):
            out_tile_desc(t).start()
            return 0

        lax.fori_loop(done, next_done, ship, 0)
        return next_done

    lax.fori_loop(0, N_EXP, expert_step, 0)

    def drain(t, _):
        out_tile_desc(0).wait()
        return 0

    lax.fori_loop(0, NK // TILE, drain, 0)


def _gmm(offs1d, x_sorted, gate_up_proj, down_proj):
    return pl.pallas_call(
        _gmm_kernel,
        in_specs=[
            pl.BlockSpec(memory_space=pltpu.SMEM),
            pl.BlockSpec(memory_space=pl.ANY),
            pl.BlockSpec(memory_space=pl.ANY),
            pl.BlockSpec(memory_space=pl.ANY),
        ],
        out_specs=pl.BlockSpec(memory_space=pl.ANY),
        out_shape=jax.ShapeDtypeStruct((NK, D_MODEL), jnp.float32),
        scratch_shapes=[
            pltpu.VMEM((NK, D_MODEL), jnp.float32),
            pltpu.VMEM((NK, D_MODEL), jnp.float32),
            pltpu.VMEM((2, D_MODEL, 2 * D_INNER), jnp.float32),
            pltpu.VMEM((2, D_INNER, D_MODEL), jnp.float32),
            pltpu.SemaphoreType.DMA,
            pltpu.SemaphoreType.DMA,
            pltpu.SemaphoreType.DMA,
            pltpu.SemaphoreType.DMA,
        ],
    )(offs1d, x_sorted, gate_up_proj, down_proj)


N_CHUNK = TOK_PER_W // CHUNK


def _combine_kernel(os_hbm, d0_hbm, d1_hbm, w0_hbm, w1_hbm, out_hbm,
                    idx0_v, idx1_v, w0_v, w1_v, r0_v, r1_v, o_v, sem_g, sem_o):
    wid = lax.axis_index("s") * 2 + lax.axis_index("c")
    tbase = wid * TOK_PER_W
    pltpu.sync_copy(d0_hbm.at[pl.ds(tbase, TOK_PER_W)], idx0_v)
    pltpu.sync_copy(d1_hbm.at[pl.ds(tbase, TOK_PER_W)], idx1_v)
    pltpu.sync_copy(w0_hbm.at[pl.ds(tbase, TOK_PER_W)], w0_v)
    pltpu.sync_copy(w1_hbm.at[pl.ds(tbase, TOK_PER_W)], w1_v)

    def gather_descs(ci, s):
        c0 = pltpu.make_async_copy(
            os_hbm.at[idx0_v.at[pl.ds(ci * CHUNK, CHUNK)]], r0_v.at[s], sem_g)
        c1 = pltpu.make_async_copy(
            os_hbm.at[idx1_v.at[pl.ds(ci * CHUNK, CHUNK)]], r1_v.at[s], sem_g)
        return c0, c1

    def out_desc(ci, s):
        return pltpu.make_async_copy(
            o_v.at[s], out_hbm.at[pl.ds(tbase + ci * CHUNK, CHUNK)], sem_o)

    g0, g1 = gather_descs(0, 0)
    g0.start()
    g1.start()

    def chunk(ci, _):
        s = lax.rem(ci, 2)

        @pl.when(ci + 1 < N_CHUNK)
        def _():
            n0, n1 = gather_descs(ci + 1, 1 - s)
            n0.start()
            n1.start()

        w0d, w1d = gather_descs(ci, s)
        w0d.wait()
        w1d.wait()

        @pl.when(ci >= 2)
        def _():
            out_desc(ci - 2, s).wait()

        def tok_group(g, _):
            wv0 = w0_v[pl.ds(ci * CHUNK + g * 16, 16)]
            wv1 = w1_v[pl.ds(ci * CHUNK + g * 16, 16)]
            for lane in range(16):
                a = wv0[lane]
                b = wv1[lane]
                t = g * 16 + lane

                @plsc.parallel_loop(0, D_MODEL // 16, unroll=8)
                def col(j, a=a, b=b, t=t, s=s):
                    sl = pl.ds(j * 16, 16)
                    o_v[s, t, sl] = a * r0_v[s, t, sl] + b * r1_v[s, t, sl]
            return 0

        lax.fori_loop(0, CHUNK // 16, tok_group, 0)
        out_desc(ci, s).start()
        return 0

    lax.fori_loop(0, N_CHUNK, chunk, 0)
    out_desc(N_CHUNK - 2, lax.rem(N_CHUNK - 2, 2)).wait()
    out_desc(N_CHUNK - 1, lax.rem(N_CHUNK - 1, 2)).wait()


@functools.lru_cache(maxsize=None)
def _combine():
    return pl.kernel(
        _combine_kernel,
        out_type=jax.ShapeDtypeStruct((N_TOK, D_MODEL), jnp.float32),
        mesh=_sc_mesh(),
        scratch_types=[
            pltpu.VMEM((TOK_PER_W,), jnp.int32),
            pltpu.VMEM((TOK_PER_W,), jnp.int32),
            pltpu.VMEM((TOK_PER_W,), jnp.float32),
            pltpu.VMEM((TOK_PER_W,), jnp.float32),
            pltpu.VMEM((2, CHUNK, D_MODEL), jnp.float32),
            pltpu.VMEM((2, CHUNK, D_MODEL), jnp.float32),
            pltpu.VMEM((2, CHUNK, D_MODEL), jnp.float32),
            pltpu.SemaphoreType.DMA,
            pltpu.SemaphoreType.DMA,
        ],
    )


def kernel(x, topk_indices, topk_weights, gate_up_proj, down_proj):
    flat2d = topk_indices.astype(jnp.int32).reshape(NK, 1)
    dest, offs = _route(flat2d)
    dest_nk = dest.reshape(N_TOK, TOP_K)
    d0 = dest_nk[:, 0]
    d1 = dest_nk[:, 1]
    w0 = topk_weights[:, 0]
    w1 = topk_weights[:, 1]
    offs1d = offs.reshape(128)

    x_sorted = _dispatch()(x, d0, d1)
    out_sorted = _gmm(offs1d, x_sorted, gate_up_proj, down_proj)
    return _combine()(out_sorted, d0, d1, w0, w1)
